# baseline (speedup 1.0000x reference)
import functools

import jax
import jax.numpy as jnp
from jax import lax
from jax.experimental import pallas as pl
from jax.experimental.pallas import tpu as pltpu
from jax.experimental.pallas import tpu_sc as plsc

N = 10000
E = 320000
HID = 128
NFEAT = 9
VOCAB = 64
NGRAPHS = 64

NC, NS = 2, 16
NW = NC * NS
CHUNK = 128
EPW = 10240
E_PAD = NW * EPW
NCH = EPW // CHUNK
N_ACC = 10240
ZROWS = 40

BLK = 1000
GRID = N // BLK

def _mesh():
    return plsc.VectorSubcoreMesh(core_axis_name="c", subcore_axis_name="s",
                                  num_cores=NC, num_subcores=NS)


def _fill_zeros(buf, nrows):
    @pl.loop(0, nrows)
    def _(i):
        for j in range(HID // 16):
            buf[i, pl.ds(j * 16, 16)] = jnp.zeros((16,), jnp.float32)


def _zero_shared(zbuf, acc, buf_rows):
    _fill_zeros(zbuf, buf_rows)
    sid = lax.axis_index("s")
    rows_per_sub = N_ACC // NS

    @pl.loop(0, rows_per_sub, step=buf_rows)
    def _(r):
        pltpu.sync_copy(zbuf, acc.at[pl.ds(sid * rows_per_sub + r, buf_rows)])


HALF = NCH // 2


HG = CHUNK // 2


def _gather_halves(g_hbm, sidx, c, buf, sem_a, sem_b, start):
    ca = g_hbm.at[sidx.at[c, pl.ds(0, HG)]], buf.at[pl.ds(0, HG)], sem_a
    cb = g_hbm.at[sidx.at[c, pl.ds(HG, HG)]], buf.at[pl.ds(HG, HG)], sem_b
    if start:
        pltpu.async_copy(*ca)
        pltpu.async_copy(*cb)
    else:
        pltpu.make_async_copy(*ca).wait()
        pltpu.make_async_copy(*cb).wait()


def _mp_body(g_hbm, src_hbm, dst_hbm, out_hbm, sidx, didx, rows0, rows1,
             acc, sem_g0, sem_g1, sem_g2, sem_g3):
    cid = lax.axis_index("c")
    sid = lax.axis_index("s")
    wid = sid * NC + cid

    _zero_shared(rows0, acc, CHUNK)
    plsc.subcore_barrier()

    for h in range(2):
        base = wid * NCH + h * HALF
        pltpu.sync_copy(src_hbm.at[pl.ds(base, HALF)], sidx)
        pltpu.sync_copy(dst_hbm.at[pl.ds(base, HALF)], didx)
        _gather_halves(g_hbm, sidx, 0, rows0, sem_g0, sem_g1, True)

        @pl.loop(0, HALF, step=2)
        def _(c):
            _gather_halves(g_hbm, sidx, c + 1, rows1, sem_g2, sem_g3, True)
            _gather_halves(g_hbm, sidx, c, rows0, sem_g0, sem_g1, False)
            pltpu.sync_copy(rows0, acc.at[didx.at[c]], add=True)

            @pl.when(c + 2 < HALF)
            def _():
                _gather_halves(g_hbm, sidx, c + 2, rows0, sem_g0, sem_g1, True)

            _gather_halves(g_hbm, sidx, c + 1, rows1, sem_g2, sem_g3, False)
            pltpu.sync_copy(rows1, acc.at[didx.at[c + 1]], add=True)

    plsc.subcore_barrier()
    orows = N_ACC // NS
    pltpu.sync_copy(acc.at[pl.ds(sid * orows, orows)],
                    out_hbm.at[cid].at[pl.ds(sid * orows, orows)])


def _message_pass(g, src2d, dst2d):
    k = pl.kernel(
        _mp_body,
        out_type=jax.ShapeDtypeStruct((NC, N_ACC, HID), jnp.float32),
        mesh=_mesh(),
        scratch_types=[
            pltpu.VMEM((HALF, CHUNK), jnp.int32),
            pltpu.VMEM((HALF, CHUNK), jnp.int32),
            pltpu.VMEM((CHUNK, HID), jnp.float32),
            pltpu.VMEM((CHUNK, HID), jnp.float32),
            pltpu.VMEM_SHARED((N_ACC, HID), jnp.float32),
            pltpu.SemaphoreType.DMA,
            pltpu.SemaphoreType.DMA,
            pltpu.SemaphoreType.DMA,
            pltpu.SemaphoreType.DMA,
        ],
    )
    return k(g, src2d, dst2d)


def _deg_body(dst_hbm, ones_hbm, out_hbm, didx, ones_v, zbuf, acc, sem):
    del sem
    cid = lax.axis_index("c")
    sid = lax.axis_index("s")
    wid = sid * NC + cid

    _zero_shared(zbuf, acc, ZROWS)
    pltpu.sync_copy(ones_hbm, ones_v)
    plsc.subcore_barrier()
    pltpu.sync_copy(dst_hbm.at[pl.ds(wid * NCH, NCH)], didx)

    @pl.loop(0, NCH)
    def _(c):
        pltpu.sync_copy(ones_v, acc.at[didx.at[c]], add=True)

    plsc.subcore_barrier()
    orows = N_ACC // NS
    pltpu.sync_copy(acc.at[pl.ds(sid * orows, orows)],
                    out_hbm.at[cid].at[pl.ds(sid * orows, orows)])


def _degrees(dst2d):
    ones128 = jnp.ones((CHUNK, HID), jnp.float32)
    k = pl.kernel(
        _deg_body,
        out_type=jax.ShapeDtypeStruct((NC, N_ACC, HID), jnp.float32),
        mesh=_mesh(),
        scratch_types=[
            pltpu.VMEM((NCH, CHUNK), jnp.int32),
            pltpu.VMEM((CHUNK, HID), jnp.float32),
            pltpu.VMEM((ZROWS, HID), jnp.float32),
            pltpu.VMEM_SHARED((N_ACC, HID), jnp.float32),
            pltpu.SemaphoreType.DMA,
        ],
    )
    return k(dst2d, ones128)


def _dinv(d0, d1):
    return lax.rsqrt(d0[:, 0:1] + d1[:, 0:1] + 1.0)


def _enc_body(x_ref, emb_ref, o_ref):
    xb = x_ref[...]
    iota = lax.broadcasted_iota(jnp.int32, (1, VOCAB), 1)
    acc = jnp.zeros((BLK, HID), jnp.float32)
    for i in range(NFEAT):
        oh = (xb[:, i:i + 1] == iota).astype(jnp.float32)
        acc = acc + jnp.dot(oh, emb_ref[i], preferred_element_type=jnp.float32)
    o_ref[...] = acc


def _encoder(x, atom_emb):
    return pl.pallas_call(
        _enc_body,
        grid=(GRID,),
        in_specs=[
            pl.BlockSpec((BLK, NFEAT), lambda i: (i, 0)),
            pl.BlockSpec((NFEAT, VOCAB, HID), lambda i: (0, 0, 0)),
        ],
        out_specs=pl.BlockSpec((BLK, HID), lambda i: (i, 0)),
        out_shape=jax.ShapeDtypeStruct((N, HID), jnp.float32),
    )(x, atom_emb)


def _pre_body(h_ref, w_ref, d0_ref, d1_ref, g_ref):
    dinv = _dinv(d0_ref[...], d1_ref[...])
    g_ref[...] = jnp.dot(h_ref[...], w_ref[...],
                         preferred_element_type=jnp.float32) * dinv


def _pre(h, W, d0, d1):
    return pl.pallas_call(
        _pre_body,
        grid=(GRID,),
        in_specs=[
            pl.BlockSpec((BLK, HID), lambda i: (i, 0)),
            pl.BlockSpec((HID, HID), lambda i: (0, 0)),
            pl.BlockSpec((BLK, HID), lambda i: (i, 0)),
            pl.BlockSpec((BLK, HID), lambda i: (i, 0)),
        ],
        out_specs=pl.BlockSpec((BLK, HID), lambda i: (i, 0)),
        out_shape=jax.ShapeDtypeStruct((N, HID), jnp.float32),
    )(h, W, d0, d1)


def _epilogue(a0, a1, g, dinv, hres, bvec, gsvec, betavec):
    conv = (a0 + a1 + g) * dinv + bvec
    z = jnp.maximum(conv, 0.0)
    z = z * gsvec + betavec
    z = jnp.maximum(z, 0.0)
    return z + hres


def _post_pre_body(a0_ref, a1_ref, g_ref, d0_ref, d1_ref, h_ref, b_ref,
                   gs_ref, beta_ref, wn_ref, h_out, g_out):
    dinv = _dinv(d0_ref[...], d1_ref[...])
    h_new = _epilogue(a0_ref[...], a1_ref[...], g_ref[...], dinv, h_ref[...],
                      b_ref[...], gs_ref[...], beta_ref[...])
    h_out[...] = h_new
    g_out[...] = jnp.dot(h_new, wn_ref[...],
                         preferred_element_type=jnp.float32) * dinv


def _post_pre(acc, g, d0, d1, h, bvec, gsvec, betavec, Wn):
    row = lambda i: (i, 0)
    return pl.pallas_call(
        _post_pre_body,
        grid=(GRID,),
        in_specs=[
            pl.BlockSpec((BLK, HID), row),
            pl.BlockSpec((BLK, HID), row),
            pl.BlockSpec((BLK, HID), row),
            pl.BlockSpec((BLK, HID), row),
            pl.BlockSpec((BLK, HID), row),
            pl.BlockSpec((BLK, HID), row),
            pl.BlockSpec((1, HID), lambda i: (0, 0)),
            pl.BlockSpec((1, HID), lambda i: (0, 0)),
            pl.BlockSpec((1, HID), lambda i: (0, 0)),
            pl.BlockSpec((HID, HID), lambda i: (0, 0)),
        ],
        out_specs=[pl.BlockSpec((BLK, HID), row), pl.BlockSpec((BLK, HID), row)],
        out_shape=[jax.ShapeDtypeStruct((N, HID), jnp.float32),
                   jax.ShapeDtypeStruct((N, HID), jnp.float32)],
    )(acc[0], acc[1], g, d0, d1, h, bvec, gsvec, betavec, Wn)


def _post_readout_body(a0_ref, a1_ref, g_ref, d0_ref, d1_ref, h_ref, b_ref,
                       gs_ref, beta_ref, bi_ref, wo_ref, bo_ref, o_ref,
                       psum, cnt):
    i = pl.program_id(0)
    dinv = _dinv(d0_ref[...], d1_ref[...])
    h_new = _epilogue(a0_ref[...], a1_ref[...], g_ref[...], dinv, h_ref[...],
                      b_ref[...], gs_ref[...], beta_ref[...])
    bidx = bi_ref[0]
    oh = (lax.broadcasted_iota(jnp.int32, (NGRAPHS, 1), 0) == bidx
          ).astype(jnp.float32)
    ps = jnp.dot(oh, h_new, preferred_element_type=jnp.float32)
    cs = jnp.sum(oh, axis=1, keepdims=True)

    @pl.when(i == 0)
    def _():
        psum[...] = ps
        cnt[...] = cs

    @pl.when(i > 0)
    def _():
        psum[...] = psum[...] + ps
        cnt[...] = cnt[...] + cs

    @pl.when(i == GRID - 1)
    def _():
        pooled = psum[...] / jnp.maximum(cnt[...], 1.0)
        logit = jnp.sum(pooled * wo_ref[...], axis=1, keepdims=True) + bo_ref[...]
        o_ref[...] = jax.nn.sigmoid(logit)


def _post_readout(acc, g, d0, d1, h, bvec, gsvec, betavec, batch3, wo_row, bo):
    row = lambda i: (i, 0)
    return pl.pallas_call(
        _post_readout_body,
        grid=(GRID,),
        in_specs=[
            pl.BlockSpec((BLK, HID), row),
            pl.BlockSpec((BLK, HID), row),
            pl.BlockSpec((BLK, HID), row),
            pl.BlockSpec((BLK, HID), row),
            pl.BlockSpec((BLK, HID), row),
            pl.BlockSpec((BLK, HID), row),
            pl.BlockSpec((1, HID), lambda i: (0, 0)),
            pl.BlockSpec((1, HID), lambda i: (0, 0)),
            pl.BlockSpec((1, HID), lambda i: (0, 0)),
            pl.BlockSpec((1, 1, BLK), lambda i: (i, 0, 0)),
            pl.BlockSpec((1, HID), lambda i: (0, 0)),
            pl.BlockSpec((1, 1), lambda i: (0, 0)),
        ],
        out_specs=pl.BlockSpec((NGRAPHS, 1), lambda i: (0, 0)),
        out_shape=jax.ShapeDtypeStruct((NGRAPHS, 1), jnp.float32),
        scratch_shapes=[pltpu.VMEM((NGRAPHS, HID), jnp.float32),
                        pltpu.VMEM((NGRAPHS, 1), jnp.float32)],
    )(acc[0], acc[1], g, d0, d1, h, bvec, gsvec, betavec, batch3, wo_row, bo)


def kernel(x, edge_index, batch_idx, atom_emb, Ws, bs, gammas, betas, W_out, b_out):
    eps = 1e-5
    gs = gammas * (1.0 / jnp.sqrt(1.0 + eps))

    pad = E_PAD - E
    srcp = jnp.concatenate([edge_index[0], jnp.zeros((pad,), jnp.int32)])
    pad_dst = N + (jnp.arange(pad, dtype=jnp.int32) % (N_ACC - N))
    dstp = jnp.concatenate([edge_index[1], pad_dst])
    src2d = srcp.reshape(-1, CHUNK)
    dst2d = dstp.reshape(-1, CHUNK)
    dst2d_deg = dst2d

    dd = _degrees(dst2d_deg)
    d0 = dd[0, :N]
    d1 = dd[1, :N]

    h = _encoder(x, atom_emb)

    nlayers = Ws.shape[0]
    g = _pre(h, Ws[0], d0, d1)
    for l in range(nlayers - 1):
        acc = _message_pass(g, src2d, dst2d)[:, :N]
        h, g = _post_pre(acc, g, d0, d1, h,
                         bs[l][None, :], gs[l][None, :], betas[l][None, :],
                         Ws[l + 1])
    acc = _message_pass(g, src2d, dst2d)[:, :N]

    lastl = nlayers - 1
    batch3 = batch_idx.reshape(GRID, 1, BLK)
    out = _post_readout(acc, g, d0, d1, h,
                        bs[lastl][None, :], gs[lastl][None, :],
                        betas[lastl][None, :],
                        batch3, W_out.reshape(1, HID), b_out.reshape(1, 1))
    return out


# speedup vs baseline: 1.0013x; 1.0013x over previous
import jax
import jax.numpy as jnp
from jax import lax
from jax.experimental import pallas as pl
from jax.experimental.pallas import tpu as pltpu
from jax.experimental.pallas import tpu_sc as plsc

N = 10000
E = 320000
HID = 128
NFEAT = 9
VOCAB = 64
NGRAPHS = 64

NC, NS = 2, 16
NW = NC * NS
CHUNK = 128
EPW = 10240
E_PAD = NW * EPW
NCH = EPW // CHUNK
N_ACC = 10240
ZROWS = 40

BLK = 1000
GRID = N // BLK

def _mesh():
    return plsc.VectorSubcoreMesh(core_axis_name="c", subcore_axis_name="s",
                                  num_cores=NC, num_subcores=NS)


def _fill_zeros(buf, nrows):
    @pl.loop(0, nrows)
    def _(i):
        for j in range(HID // 16):
            buf[i, pl.ds(j * 16, 16)] = jnp.zeros((16,), jnp.float32)


def _zero_shared(zbuf, acc, buf_rows):
    _fill_zeros(zbuf, buf_rows)
    sid = lax.axis_index("s")
    rows_per_sub = N_ACC // NS

    @pl.loop(0, rows_per_sub, step=buf_rows)
    def _(r):
        pltpu.sync_copy(zbuf, acc.at[pl.ds(sid * rows_per_sub + r, buf_rows)])


HALF = NCH // 2


def _mp_body(g_hbm, src_hbm, dst_hbm, out_hbm, sidx, didx, rows0, rows1,
             acc, sem_g0, sem_g1):
    cid = lax.axis_index("c")
    sid = lax.axis_index("s")
    wid = sid * NC + cid

    _zero_shared(rows0, acc, CHUNK)
    plsc.subcore_barrier()

    for h in range(2):
        base = wid * NCH + h * HALF
        pltpu.sync_copy(src_hbm.at[pl.ds(base, HALF)], sidx)
        pltpu.sync_copy(dst_hbm.at[pl.ds(base, HALF)], didx)
        pltpu.async_copy(g_hbm.at[sidx.at[0]], rows0, sem_g0)

        @pl.loop(0, HALF, step=2)
        def _(c):
            pltpu.async_copy(g_hbm.at[sidx.at[c + 1]], rows1, sem_g1)
            pltpu.make_async_copy(g_hbm.at[sidx.at[c]], rows0, sem_g0).wait()
            pltpu.sync_copy(rows0, acc.at[didx.at[c]], add=True)

            @pl.when(c + 2 < HALF)
            def _():
                pltpu.async_copy(g_hbm.at[sidx.at[c + 2]], rows0, sem_g0)

            pltpu.make_async_copy(g_hbm.at[sidx.at[c + 1]], rows1, sem_g1).wait()
            pltpu.sync_copy(rows1, acc.at[didx.at[c + 1]], add=True)

    plsc.subcore_barrier()
    orows = N_ACC // NS
    pltpu.sync_copy(acc.at[pl.ds(sid * orows, orows)],
                    out_hbm.at[cid].at[pl.ds(sid * orows, orows)])


def _message_pass(g, src2d, dst2d):
    k = pl.kernel(
        _mp_body,
        out_type=jax.ShapeDtypeStruct((NC, N_ACC, HID), jnp.float32),
        mesh=_mesh(),
        scratch_types=[
            pltpu.VMEM((HALF, CHUNK), jnp.int32),
            pltpu.VMEM((HALF, CHUNK), jnp.int32),
            pltpu.VMEM((CHUNK, HID), jnp.float32),
            pltpu.VMEM((CHUNK, HID), jnp.float32),
            pltpu.VMEM_SHARED((N_ACC, HID), jnp.float32),
            pltpu.SemaphoreType.DMA,
            pltpu.SemaphoreType.DMA,
        ],
    )
    return k(g, src2d, dst2d)


def _deg_body(dst_hbm, ones_hbm, out_hbm, didx, ones_v, zbuf, acc, sem):
    del sem
    cid = lax.axis_index("c")
    sid = lax.axis_index("s")
    wid = sid * NC + cid

    _zero_shared(zbuf, acc, ZROWS)
    pltpu.sync_copy(ones_hbm, ones_v)
    plsc.subcore_barrier()
    pltpu.sync_copy(dst_hbm.at[pl.ds(wid * NCH, NCH)], didx)

    @pl.loop(0, NCH)
    def _(c):
        pltpu.sync_copy(ones_v, acc.at[didx.at[c]], add=True)

    plsc.subcore_barrier()
    orows = N_ACC // NS
    pltpu.sync_copy(acc.at[pl.ds(sid * orows, orows)],
                    out_hbm.at[cid].at[pl.ds(sid * orows, orows)])


def _degrees(dst2d):
    ones128 = jnp.ones((CHUNK, HID), jnp.float32)
    k = pl.kernel(
        _deg_body,
        out_type=jax.ShapeDtypeStruct((NC, N_ACC, HID), jnp.float32),
        mesh=_mesh(),
        scratch_types=[
            pltpu.VMEM((NCH, CHUNK), jnp.int32),
            pltpu.VMEM((CHUNK, HID), jnp.float32),
            pltpu.VMEM((ZROWS, HID), jnp.float32),
            pltpu.VMEM_SHARED((N_ACC, HID), jnp.float32),
            pltpu.SemaphoreType.DMA,
        ],
    )
    return k(dst2d, ones128)


def _dinv(d0, d1):
    return lax.rsqrt(d0[:, 0:1] + d1[:, 0:1] + 1.0)


def _enc_body(x_ref, emb_ref, o_ref):
    xb = x_ref[...]
    iota = lax.broadcasted_iota(jnp.int32, (1, VOCAB), 1)
    acc = jnp.zeros((BLK, HID), jnp.float32)
    for i in range(NFEAT):
        oh = (xb[:, i:i + 1] == iota).astype(jnp.float32)
        acc = acc + jnp.dot(oh, emb_ref[i], preferred_element_type=jnp.float32)
    o_ref[...] = acc


def _encoder(x, atom_emb):
    return pl.pallas_call(
        _enc_body,
        grid=(GRID,),
        in_specs=[
            pl.BlockSpec((BLK, NFEAT), lambda i: (i, 0)),
            pl.BlockSpec((NFEAT, VOCAB, HID), lambda i: (0, 0, 0)),
        ],
        out_specs=pl.BlockSpec((BLK, HID), lambda i: (i, 0)),
        out_shape=jax.ShapeDtypeStruct((N, HID), jnp.float32),
    )(x, atom_emb)


def _pre_body(h_ref, w_ref, d0_ref, d1_ref, g_ref):
    dinv = _dinv(d0_ref[...], d1_ref[...])
    g_ref[...] = jnp.dot(h_ref[...], w_ref[...],
                         preferred_element_type=jnp.float32) * dinv


def _pre(h, W, d0, d1):
    return pl.pallas_call(
        _pre_body,
        grid=(GRID,),
        in_specs=[
            pl.BlockSpec((BLK, HID), lambda i: (i, 0)),
            pl.BlockSpec((HID, HID), lambda i: (0, 0)),
            pl.BlockSpec((BLK, HID), lambda i: (i, 0)),
            pl.BlockSpec((BLK, HID), lambda i: (i, 0)),
        ],
        out_specs=pl.BlockSpec((BLK, HID), lambda i: (i, 0)),
        out_shape=jax.ShapeDtypeStruct((N, HID), jnp.float32),
    )(h, W, d0, d1)


def _epilogue(a0, a1, g, dinv, hres, bvec, gsvec, betavec):
    conv = (a0 + a1 + g) * dinv + bvec
    z = jnp.maximum(conv, 0.0)
    z = z * gsvec + betavec
    z = jnp.maximum(z, 0.0)
    return z + hres


def _post_pre_body(a0_ref, a1_ref, g_ref, d0_ref, d1_ref, h_ref, b_ref,
                   gs_ref, beta_ref, wn_ref, h_out, g_out):
    dinv = _dinv(d0_ref[...], d1_ref[...])
    h_new = _epilogue(a0_ref[...], a1_ref[...], g_ref[...], dinv, h_ref[...],
                      b_ref[...], gs_ref[...], beta_ref[...])
    h_out[...] = h_new
    g_out[...] = jnp.dot(h_new, wn_ref[...],
                         preferred_element_type=jnp.float32) * dinv


def _post_pre(acc, g, d0, d1, h, bvec, gsvec, betavec, Wn):
    row = lambda i: (i, 0)
    return pl.pallas_call(
        _post_pre_body,
        grid=(GRID,),
        in_specs=[
            pl.BlockSpec((BLK, HID), row),
            pl.BlockSpec((BLK, HID), row),
            pl.BlockSpec((BLK, HID), row),
            pl.BlockSpec((BLK, HID), row),
            pl.BlockSpec((BLK, HID), row),
            pl.BlockSpec((BLK, HID), row),
            pl.BlockSpec((1, HID), lambda i: (0, 0)),
            pl.BlockSpec((1, HID), lambda i: (0, 0)),
            pl.BlockSpec((1, HID), lambda i: (0, 0)),
            pl.BlockSpec((HID, HID), lambda i: (0, 0)),
        ],
        out_specs=[pl.BlockSpec((BLK, HID), row), pl.BlockSpec((BLK, HID), row)],
        out_shape=[jax.ShapeDtypeStruct((N, HID), jnp.float32),
                   jax.ShapeDtypeStruct((N, HID), jnp.float32)],
    )(acc[0], acc[1], g, d0, d1, h, bvec, gsvec, betavec, Wn)


def _post_readout_body(a0_ref, a1_ref, g_ref, d0_ref, d1_ref, h_ref, b_ref,
                       gs_ref, beta_ref, bi_ref, wo_ref, bo_ref, o_ref,
                       psum, cnt):
    i = pl.program_id(0)
    dinv = _dinv(d0_ref[...], d1_ref[...])
    h_new = _epilogue(a0_ref[...], a1_ref[...], g_ref[...], dinv, h_ref[...],
                      b_ref[...], gs_ref[...], beta_ref[...])
    bidx = bi_ref[0]
    oh = (lax.broadcasted_iota(jnp.int32, (NGRAPHS, 1), 0) == bidx
          ).astype(jnp.float32)
    ps = jnp.dot(oh, h_new, preferred_element_type=jnp.float32)
    cs = jnp.sum(oh, axis=1, keepdims=True)

    @pl.when(i == 0)
    def _():
        psum[...] = ps
        cnt[...] = cs

    @pl.when(i > 0)
    def _():
        psum[...] = psum[...] + ps
        cnt[...] = cnt[...] + cs

    @pl.when(i == GRID - 1)
    def _():
        pooled = psum[...] / jnp.maximum(cnt[...], 1.0)
        logit = jnp.sum(pooled * wo_ref[...], axis=1, keepdims=True) + bo_ref[...]
        o_ref[...] = jax.nn.sigmoid(logit)


def _post_readout(acc, g, d0, d1, h, bvec, gsvec, betavec, batch3, wo_row, bo):
    row = lambda i: (i, 0)
    return pl.pallas_call(
        _post_readout_body,
        grid=(GRID,),
        in_specs=[
            pl.BlockSpec((BLK, HID), row),
            pl.BlockSpec((BLK, HID), row),
            pl.BlockSpec((BLK, HID), row),
            pl.BlockSpec((BLK, HID), row),
            pl.BlockSpec((BLK, HID), row),
            pl.BlockSpec((BLK, HID), row),
            pl.BlockSpec((1, HID), lambda i: (0, 0)),
            pl.BlockSpec((1, HID), lambda i: (0, 0)),
            pl.BlockSpec((1, HID), lambda i: (0, 0)),
            pl.BlockSpec((1, 1, BLK), lambda i: (i, 0, 0)),
            pl.BlockSpec((1, HID), lambda i: (0, 0)),
            pl.BlockSpec((1, 1), lambda i: (0, 0)),
        ],
        out_specs=pl.BlockSpec((NGRAPHS, 1), lambda i: (0, 0)),
        out_shape=jax.ShapeDtypeStruct((NGRAPHS, 1), jnp.float32),
        scratch_shapes=[pltpu.VMEM((NGRAPHS, HID), jnp.float32),
                        pltpu.VMEM((NGRAPHS, 1), jnp.float32)],
    )(acc[0], acc[1], g, d0, d1, h, bvec, gsvec, betavec, batch3, wo_row, bo)


def kernel(x, edge_index, batch_idx, atom_emb, Ws, bs, gammas, betas, W_out, b_out):
    eps = 1e-5
    gs = gammas * (1.0 / jnp.sqrt(1.0 + eps))

    pad = E_PAD - E
    srcp = jnp.concatenate([edge_index[0], jnp.zeros((pad,), jnp.int32)])
    pad_dst = N + (jnp.arange(pad, dtype=jnp.int32) % (N_ACC - N))
    dstp = jnp.concatenate([edge_index[1], pad_dst])
    src2d = srcp.reshape(-1, CHUNK)
    dst2d = dstp.reshape(-1, CHUNK)
    dst2d_deg = dst2d

    dd = _degrees(dst2d_deg)
    d0 = dd[0, :N]
    d1 = dd[1, :N]

    h = _encoder(x, atom_emb)

    nlayers = Ws.shape[0]
    g = _pre(h, Ws[0], d0, d1)
    for l in range(nlayers - 1):
        acc = _message_pass(g, src2d, dst2d)[:, :N]
        h, g = _post_pre(acc, g, d0, d1, h,
                         bs[l][None, :], gs[l][None, :], betas[l][None, :],
                         Ws[l + 1])
    acc = _message_pass(g, src2d, dst2d)[:, :N]

    lastl = nlayers - 1
    batch3 = batch_idx.reshape(GRID, 1, BLK)
    out = _post_readout(acc, g, d0, d1, h,
                        bs[lastl][None, :], gs[lastl][None, :],
                        betas[lastl][None, :],
                        batch3, W_out.reshape(1, HID), b_out.reshape(1, 1))
    return out
